# initial kernel scaffold (unmeasured)
import math

import jax
import jax.numpy as jnp
from jax import lax
from jax.experimental import pallas as pl
from jax.experimental.pallas import tpu as pltpu

N_DEV = 4
SQ = 1024
D = 1024
HQ = 8
DH = 128
SCALE = 0.08838834764831843


def kernel(x, Wq, Wk, Wv, Wo):
    def body(x_hbm, wq_hbm, wk_hbm, wv_hbm, wo_hbm, out_ref,
             stage, xbf, wq3, wk3, wv3, wo3, cosb, sinb, acc,
             xrecv, psend, precv,
             stage_sem, xsend_sems, xrecv_sems, psend_sems, precv_sems):
        i = lax.axis_index("i")
        bf16 = jnp.bfloat16

        cp_x = pltpu.make_async_copy(x_hbm.at[0], stage, stage_sem)
        cp_x.start()

        barrier = pltpu.get_barrier_semaphore()
        for s in (1, 2, 3):
            pl.semaphore_signal(
                barrier, inc=1,
                device_id=((i + s) % N_DEV,),
                device_id_type=pl.DeviceIdType.MESH,
            )
        pl.semaphore_wait(barrier, 3)

        cp_x.wait()
        xbf[...] = stage[...].astype(bf16)

        x_rdmas = []
        for s in (2, 1, 3):
            rdma = pltpu.make_async_remote_copy(
                src_ref=xbf,
                dst_ref=xrecv.at[3 - s],
                send_sem=xsend_sems.at[s - 1],
                recv_sem=xrecv_sems.at[3 - s],
                device_id=((i + s) % N_DEV,),
                device_id_type=pl.DeviceIdType.MESH,
            )
            rdma.start()
            x_rdmas.append(rdma)

        for w_hbm, w3, by_rows in ((wq_hbm, wq3, False), (wk_hbm, wk3, False),
                                   (wv_hbm, wv3, False), (wo_hbm, wo3, True)):
            cp_w = pltpu.make_async_copy(w_hbm, stage, stage_sem)
            cp_w.start()
            cp_w.wait()
            for h in range(HQ):
                if by_rows:
                    w3[h] = stage[h * DH:(h + 1) * DH, :].astype(bf16)
                else:
                    w3[h] = stage[:, h * DH:(h + 1) * DH].astype(bf16)

        jcol = lax.broadcasted_iota(jnp.int32, (SQ, DH), 1)
        pos = lax.broadcasted_iota(jnp.float32, (SQ, DH), 0)
        expnt = (jcol - (jcol % 2)).astype(jnp.float32) * (1.0 / DH)
        ang = pos * jnp.exp(-math.log(10000.0) * expnt)
        cosb[...] = jnp.cos(ang)
        sinb[...] = jnp.sin(ang)
        even = (jcol % 2) == 0

        def rope(t):
            t_r = jnp.where(even,
                            -jnp.roll(t, -1, axis=1),
                            jnp.roll(t, 1, axis=1))
            return t * cosb[...] + t_r * sinb[...]

        def compute_partial(xb):
            acc[...] = jnp.zeros((SQ, D), jnp.float32)

            def head(h, carry):
                q = jnp.dot(xb, wq3[h], preferred_element_type=jnp.float32)
                k = jnp.dot(xb, wk3[h], preferred_element_type=jnp.float32)
                v = jnp.dot(xb, wv3[h], preferred_element_type=jnp.float32)
                q = (rope(q) * SCALE).astype(bf16)
                k = rope(k).astype(bf16)
                s = lax.dot_general(q, k, (((1,), (1,)), ((), ())),
                                    preferred_element_type=jnp.float32)
                m = jnp.max(s, axis=1, keepdims=True)
                w = jnp.exp(s - m)
                w = (w / jnp.sum(w, axis=1, keepdims=True)).astype(bf16)
                ctx = jnp.dot(w, v.astype(bf16),
                              preferred_element_type=jnp.float32)
                acc[...] += jnp.dot(ctx.astype(bf16), wo3[h],
                                    preferred_element_type=jnp.float32)
                return carry

            lax.fori_loop(0, HQ, head, 0)

        compute_partial(xbf[...])
        out_ref[0] = acc[...]

        p_rdmas = []
        for s in (1, 3, 2):
            k = s - 1
            pltpu.make_async_remote_copy(
                src_ref=xbf, dst_ref=xrecv.at[k],
                send_sem=xsend_sems.at[k], recv_sem=xrecv_sems.at[k],
                device_id=(i, ), device_id_type=pl.DeviceIdType.MESH,
            ).wait_recv()
            compute_partial(xrecv[k])
            psend[k] = acc[...].astype(bf16)
            rdma = pltpu.make_async_remote_copy(
                src_ref=psend.at[k],
                dst_ref=precv.at[3 - s],
                send_sem=psend_sems.at[k],
                recv_sem=precv_sems.at[3 - s],
                device_id=((i + s) % N_DEV,),
                device_id_type=pl.DeviceIdType.MESH,
            )
            rdma.start()
            p_rdmas.append(rdma)

        for k in range(3):
            pltpu.make_async_remote_copy(
                src_ref=psend.at[k], dst_ref=precv.at[k],
                send_sem=psend_sems.at[k], recv_sem=precv_sems.at[k],
                device_id=(i,), device_id_type=pl.DeviceIdType.MESH,
            ).wait_recv()
        out_ref[0] = (out_ref[0]
                      + precv[0].astype(jnp.float32)
                      + precv[1].astype(jnp.float32)
                      + precv[2].astype(jnp.float32))

        for rdma in x_rdmas + p_rdmas:
            rdma.wait_send()

    return pl.pallas_call(
        body,
        out_shape=jax.ShapeDtypeStruct((1, SQ, D), jnp.float32),
        in_specs=[pl.BlockSpec(memory_space=pltpu.ANY)] * 5,
        out_specs=pl.BlockSpec(memory_space=pltpu.VMEM),
        scratch_shapes=[
            pltpu.VMEM((SQ, D), jnp.float32),
            pltpu.VMEM((SQ, D), jnp.bfloat16),
            pltpu.VMEM((HQ, D, DH), jnp.bfloat16),
            pltpu.VMEM((HQ, D, DH), jnp.bfloat16),
            pltpu.VMEM((HQ, D, DH), jnp.bfloat16),
            pltpu.VMEM((HQ, DH, D), jnp.bfloat16),
            pltpu.VMEM((SQ, DH), jnp.float32),
            pltpu.VMEM((SQ, DH), jnp.float32),
            pltpu.VMEM((SQ, D), jnp.float32),
            pltpu.VMEM((3, SQ, D), jnp.bfloat16),
            pltpu.VMEM((3, SQ, D), jnp.bfloat16),
            pltpu.VMEM((3, SQ, D), jnp.bfloat16),
            pltpu.SemaphoreType.DMA,
            pltpu.SemaphoreType.DMA((3,)),
            pltpu.SemaphoreType.DMA((3,)),
            pltpu.SemaphoreType.DMA((3,)),
            pltpu.SemaphoreType.DMA((3,)),
        ],
        compiler_params=pltpu.CompilerParams(collective_id=0),
    )(x, Wq, Wk, Wv, Wo)


# baseline (device time: 220769 ns/iter reference)
import math

import jax
import jax.numpy as jnp
from jax import lax
from jax.experimental import pallas as pl
from jax.experimental.pallas import tpu as pltpu

N_DEV = 4
SQ = 1024
D = 1024
HQ = 8
DH = 128
SCALE = 0.08838834764831843


def kernel(x, Wq, Wk, Wv, Wo):
    def body(x_hbm, wq_hbm, wk_hbm, wv_hbm, wo_hbm, out_ref,
             stage, xbf, wq3, wk3, wv3, wo3, cosb, sinb,
             xrecv, precv,
             stage_sem, xsend_sems, xrecv_sems, psend_sems, precv_sems):
        i = lax.axis_index("i")
        bf16 = jnp.bfloat16

        cp_x = pltpu.make_async_copy(x_hbm.at[0], stage, stage_sem)
        cp_x.start()

        barrier = pltpu.get_barrier_semaphore()
        for s in (1, 2, 3):
            pl.semaphore_signal(
                barrier, inc=1,
                device_id=((i + s) % N_DEV,),
                device_id_type=pl.DeviceIdType.MESH,
            )
        pl.semaphore_wait(barrier, 3)

        cp_x.wait()
        xbf[...] = stage[...].astype(bf16)

        x_rdmas = []
        for s in (2, 1, 3):
            rdma = pltpu.make_async_remote_copy(
                src_ref=xbf,
                dst_ref=xrecv.at[3 - s],
                send_sem=xsend_sems.at[s - 1],
                recv_sem=xrecv_sems.at[3 - s],
                device_id=((i + s) % N_DEV,),
                device_id_type=pl.DeviceIdType.MESH,
            )
            rdma.start()
            x_rdmas.append(rdma)

        for w_hbm, w3, by_rows in ((wq_hbm, wq3, False), (wk_hbm, wk3, False),
                                   (wv_hbm, wv3, False), (wo_hbm, wo3, True)):
            cp_w = pltpu.make_async_copy(w_hbm, stage, stage_sem)
            cp_w.start()
            cp_w.wait()
            for h in range(HQ):
                if by_rows:
                    w3[h] = stage[h * DH:(h + 1) * DH, :].astype(bf16)
                else:
                    w3[h] = stage[:, h * DH:(h + 1) * DH].astype(bf16)

        jcol = lax.broadcasted_iota(jnp.int32, (SQ, DH), 1)
        pos = lax.broadcasted_iota(jnp.int32, (SQ, DH), 0).astype(jnp.float32)
        expnt = (jcol - (jcol % 2)).astype(jnp.float32) * (1.0 / DH)
        ang = pos * jnp.exp(-math.log(10000.0) * expnt)
        cosb[...] = jnp.cos(ang)
        sinb[...] = jnp.sin(ang)
        even = (jcol % 2) == 0

        acc = stage
        psend = xrecv

        def rope(t):
            t_r = jnp.where(even,
                            -jnp.roll(t, -1, axis=1),
                            jnp.roll(t, 1, axis=1))
            return t * cosb[...] + t_r * sinb[...]

        def compute_partial(xb):
            acc[...] = jnp.zeros((SQ, D), jnp.float32)

            def head(h, carry):
                q = jnp.dot(xb, wq3[h], preferred_element_type=jnp.float32)
                k = jnp.dot(xb, wk3[h], preferred_element_type=jnp.float32)
                v = jnp.dot(xb, wv3[h], preferred_element_type=jnp.float32)
                q = (rope(q) * SCALE).astype(bf16)
                k = rope(k).astype(bf16)
                s = lax.dot_general(q, k, (((1,), (1,)), ((), ())),
                                    preferred_element_type=jnp.float32)
                m = jnp.max(s, axis=1, keepdims=True)
                w = jnp.exp(s - m)
                w = (w / jnp.sum(w, axis=1, keepdims=True)).astype(bf16)
                ctx = jnp.dot(w, v.astype(bf16),
                              preferred_element_type=jnp.float32)
                acc[...] += jnp.dot(ctx.astype(bf16), wo3[h],
                                    preferred_element_type=jnp.float32)
                return carry

            lax.fori_loop(0, HQ, head, 0)

        p_rdmas = []
        for s in (1, 3, 2):
            k = s - 1
            pltpu.make_async_remote_copy(
                src_ref=xbf, dst_ref=xrecv.at[k],
                send_sem=xsend_sems.at[k], recv_sem=xrecv_sems.at[k],
                device_id=(i, ), device_id_type=pl.DeviceIdType.MESH,
            ).wait_recv()
            compute_partial(xrecv[k])
            psend[k] = acc[...].astype(bf16)
            rdma = pltpu.make_async_remote_copy(
                src_ref=psend.at[k],
                dst_ref=precv.at[3 - s],
                send_sem=psend_sems.at[k],
                recv_sem=precv_sems.at[3 - s],
                device_id=((i + s) % N_DEV,),
                device_id_type=pl.DeviceIdType.MESH,
            )
            rdma.start()
            p_rdmas.append(rdma)

        compute_partial(xbf[...])

        for k in range(3):
            pltpu.make_async_remote_copy(
                src_ref=psend.at[k], dst_ref=precv.at[k],
                send_sem=psend_sems.at[k], recv_sem=precv_sems.at[k],
                device_id=(i,), device_id_type=pl.DeviceIdType.MESH,
            ).wait_recv()
        acc[...] = (acc[...]
                    + precv[0].astype(jnp.float32)
                    + precv[1].astype(jnp.float32)
                    + precv[2].astype(jnp.float32))

        cp_out = pltpu.make_async_copy(acc, out_ref.at[0], stage_sem)
        cp_out.start()
        cp_out.wait()

        for rdma in x_rdmas + p_rdmas:
            rdma.wait_send()

    return pl.pallas_call(
        body,
        out_shape=jax.ShapeDtypeStruct((1, SQ, D), jnp.float32),
        in_specs=[pl.BlockSpec(memory_space=pl.ANY)] * 5,
        out_specs=pl.BlockSpec(memory_space=pltpu.MemorySpace.HBM),
        scratch_shapes=[
            pltpu.VMEM((SQ, D), jnp.float32),
            pltpu.VMEM((SQ, D), jnp.bfloat16),
            pltpu.VMEM((HQ, D, DH), jnp.bfloat16),
            pltpu.VMEM((HQ, D, DH), jnp.bfloat16),
            pltpu.VMEM((HQ, D, DH), jnp.bfloat16),
            pltpu.VMEM((HQ, DH, D), jnp.bfloat16),
            pltpu.VMEM((SQ, DH), jnp.float32),
            pltpu.VMEM((SQ, DH), jnp.float32),
            pltpu.VMEM((3, SQ, D), jnp.bfloat16),
            pltpu.VMEM((3, SQ, D), jnp.bfloat16),
            pltpu.SemaphoreType.DMA,
            pltpu.SemaphoreType.DMA((3,)),
            pltpu.SemaphoreType.DMA((3,)),
            pltpu.SemaphoreType.DMA((3,)),
            pltpu.SemaphoreType.DMA((3,)),
        ],
        compiler_params=pltpu.CompilerParams(
            collective_id=0,
            vmem_limit_bytes=100 * 1024 * 1024,
        ),
    )(x, Wq, Wk, Wv, Wo)


# device time: 179532 ns/iter; 1.2297x vs baseline; 1.2297x over previous
import math

import jax
import jax.numpy as jnp
from jax import lax
from jax.experimental import pallas as pl
from jax.experimental.pallas import tpu as pltpu

N_DEV = 4
SQ = 1024
D = 1024
HQ = 8
DH = 128
SCALE = 0.08838834764831843


def kernel(x, Wq, Wk, Wv, Wo):
    def body(x_hbm, wq_hbm, wk_hbm, wv_hbm, wo_hbm, out_ref,
             stage, xbf, wqb, wkb, wvb, wo3, cosb, sinb,
             q3, k3, v3, xrecv, precv,
             stage_sem, xsend_sems, xrecv_sems, psend_sems, precv_sems):
        i = lax.axis_index("i")
        bf16 = jnp.bfloat16

        cp_x = pltpu.make_async_copy(x_hbm.at[0], stage, stage_sem)
        cp_x.start()

        barrier = pltpu.get_barrier_semaphore()
        for s in (1, 2, 3):
            pl.semaphore_signal(
                barrier, inc=1,
                device_id=((i + s) % N_DEV,),
                device_id_type=pl.DeviceIdType.MESH,
            )
        pl.semaphore_wait(barrier, 3)

        cp_x.wait()
        xbf[...] = stage[...].astype(bf16)

        x_rdmas = []
        for s in (2, 1, 3):
            rdma = pltpu.make_async_remote_copy(
                src_ref=xbf,
                dst_ref=xrecv.at[3 - s],
                send_sem=xsend_sems.at[s - 1],
                recv_sem=xrecv_sems.at[3 - s],
                device_id=((i + s) % N_DEV,),
                device_id_type=pl.DeviceIdType.MESH,
            )
            rdma.start()
            x_rdmas.append(rdma)

        for w_hbm, wb in ((wq_hbm, wqb), (wk_hbm, wkb), (wv_hbm, wvb)):
            cp_w = pltpu.make_async_copy(w_hbm, stage, stage_sem)
            cp_w.start()
            cp_w.wait()
            wb[...] = stage[...].astype(bf16)
        cp_w = pltpu.make_async_copy(wo_hbm, stage, stage_sem)
        cp_w.start()
        cp_w.wait()
        for h in range(HQ):
            wo3[h] = stage[h * DH:(h + 1) * DH, :].astype(bf16)

        jcol = lax.broadcasted_iota(jnp.int32, (SQ, D), 1)
        pos = lax.broadcasted_iota(jnp.int32, (SQ, D), 0).astype(jnp.float32)
        d_in_head = jcol % DH
        expnt = (d_in_head - (d_in_head % 2)).astype(jnp.float32) * (1.0 / DH)
        ang = pos * jnp.exp(-math.log(10000.0) * expnt)
        cosb[...] = jnp.cos(ang)
        sinb[...] = jnp.sin(ang)
        even = (jcol % 2) == 0

        acc = stage
        psend = xrecv

        def rope(t):
            t_r = jnp.where(even,
                            -jnp.roll(t, -1, axis=1),
                            jnp.roll(t, 1, axis=1))
            return t * cosb[...] + t_r * sinb[...]

        def compute_partial(xb):
            qf = (rope(jnp.dot(xb, wqb[...],
                               preferred_element_type=jnp.float32))
                  * SCALE).astype(bf16)
            kf = rope(jnp.dot(xb, wkb[...],
                              preferred_element_type=jnp.float32)).astype(bf16)
            vf = jnp.dot(xb, wvb[...],
                         preferred_element_type=jnp.float32).astype(bf16)
            for h in range(HQ):
                q3[h] = qf[:, h * DH:(h + 1) * DH]
                k3[h] = kf[:, h * DH:(h + 1) * DH]
                v3[h] = vf[:, h * DH:(h + 1) * DH]

            acc[...] = jnp.zeros((SQ, D), jnp.float32)

            def head(h, carry):
                s = lax.dot_general(q3[h], k3[h], (((1,), (1,)), ((), ())),
                                    preferred_element_type=jnp.float32)
                w = jnp.exp(s)
                denom = jnp.sum(w, axis=1, keepdims=True)
                ctx = jnp.dot(w.astype(bf16), v3[h],
                              preferred_element_type=jnp.float32) / denom
                acc[...] += jnp.dot(ctx.astype(bf16), wo3[h],
                                    preferred_element_type=jnp.float32)
                return carry

            lax.fori_loop(0, HQ, head, 0)

        p_rdmas = []
        for s in (1, 3, 2):
            k = s - 1
            pltpu.make_async_remote_copy(
                src_ref=xbf, dst_ref=xrecv.at[k],
                send_sem=xsend_sems.at[k], recv_sem=xrecv_sems.at[k],
                device_id=(i, ), device_id_type=pl.DeviceIdType.MESH,
            ).wait_recv()
            compute_partial(xrecv[k])
            psend[k] = acc[...].astype(bf16)
            rdma = pltpu.make_async_remote_copy(
                src_ref=psend.at[k],
                dst_ref=precv.at[3 - s],
                send_sem=psend_sems.at[k],
                recv_sem=precv_sems.at[3 - s],
                device_id=((i + s) % N_DEV,),
                device_id_type=pl.DeviceIdType.MESH,
            )
            rdma.start()
            p_rdmas.append(rdma)

        compute_partial(xbf[...])

        for k in range(3):
            pltpu.make_async_remote_copy(
                src_ref=psend.at[k], dst_ref=precv.at[k],
                send_sem=psend_sems.at[k], recv_sem=precv_sems.at[k],
                device_id=(i,), device_id_type=pl.DeviceIdType.MESH,
            ).wait_recv()
        acc[...] = (acc[...]
                    + precv[0].astype(jnp.float32)
                    + precv[1].astype(jnp.float32)
                    + precv[2].astype(jnp.float32))

        cp_out = pltpu.make_async_copy(acc, out_ref.at[0], stage_sem)
        cp_out.start()
        cp_out.wait()

        for rdma in x_rdmas + p_rdmas:
            rdma.wait_send()

    return pl.pallas_call(
        body,
        out_shape=jax.ShapeDtypeStruct((1, SQ, D), jnp.float32),
        in_specs=[pl.BlockSpec(memory_space=pl.ANY)] * 5,
        out_specs=pl.BlockSpec(memory_space=pltpu.MemorySpace.HBM),
        scratch_shapes=[
            pltpu.VMEM((SQ, D), jnp.float32),
            pltpu.VMEM((SQ, D), jnp.bfloat16),
            pltpu.VMEM((SQ, D), jnp.bfloat16),
            pltpu.VMEM((SQ, D), jnp.bfloat16),
            pltpu.VMEM((SQ, D), jnp.bfloat16),
            pltpu.VMEM((HQ, DH, D), jnp.bfloat16),
            pltpu.VMEM((SQ, D), jnp.float32),
            pltpu.VMEM((SQ, D), jnp.float32),
            pltpu.VMEM((HQ, SQ, DH), jnp.bfloat16),
            pltpu.VMEM((HQ, SQ, DH), jnp.bfloat16),
            pltpu.VMEM((HQ, SQ, DH), jnp.bfloat16),
            pltpu.VMEM((3, SQ, D), jnp.bfloat16),
            pltpu.VMEM((3, SQ, D), jnp.bfloat16),
            pltpu.SemaphoreType.DMA,
            pltpu.SemaphoreType.DMA((3,)),
            pltpu.SemaphoreType.DMA((3,)),
            pltpu.SemaphoreType.DMA((3,)),
            pltpu.SemaphoreType.DMA((3,)),
        ],
        compiler_params=pltpu.CompilerParams(
            collective_id=0,
            vmem_limit_bytes=100 * 1024 * 1024,
        ),
    )(x, Wq, Wk, Wv, Wo)


# device time: 166689 ns/iter; 1.3244x vs baseline; 1.0770x over previous
import math

import jax
import jax.numpy as jnp
from jax import lax
from jax.experimental import pallas as pl
from jax.experimental.pallas import tpu as pltpu

N_DEV = 4
SQ = 1024
D = 1024
HQ = 8
DH = 128
SCALE = 0.08838834764831843


def kernel(x, Wq, Wk, Wv, Wo):
    def body(x_hbm, wq_hbm, wk_hbm, wv_hbm, wo_hbm, out_ref,
             stage, xbf, wqb, wkb, wvb, wo3, cosb, sinb,
             q3, k3, v3, xrecv, precv,
             stage_sem, xsend_sems, xrecv_sems, psend_sems, precv_sems):
        i = lax.axis_index("i")
        bf16 = jnp.bfloat16

        cp_x = pltpu.make_async_copy(x_hbm.at[0], stage, stage_sem)
        cp_x.start()

        barrier = pltpu.get_barrier_semaphore()
        for s in (1, 2, 3):
            pl.semaphore_signal(
                barrier, inc=1,
                device_id=((i + s) % N_DEV,),
                device_id_type=pl.DeviceIdType.MESH,
            )
        pl.semaphore_wait(barrier, 3)

        cp_x.wait()
        xbf[...] = stage[...].astype(bf16)

        x_rdmas = []
        for s in (2, 1, 3):
            rdma = pltpu.make_async_remote_copy(
                src_ref=xbf,
                dst_ref=xrecv.at[3 - s],
                send_sem=xsend_sems.at[s - 1],
                recv_sem=xrecv_sems.at[3 - s],
                device_id=((i + s) % N_DEV,),
                device_id_type=pl.DeviceIdType.MESH,
            )
            rdma.start()
            x_rdmas.append(rdma)

        for w_hbm, wb, scl in ((wq_hbm, wqb, SCALE), (wk_hbm, wkb, 1.0),
                               (wv_hbm, wvb, 1.0)):
            cp_w = pltpu.make_async_copy(w_hbm, stage, stage_sem)
            cp_w.start()
            cp_w.wait()
            wb[...] = (stage[...] * scl).astype(bf16)
        cp_w = pltpu.make_async_copy(wo_hbm, stage, stage_sem)
        cp_w.start()
        cp_w.wait()
        for h in range(HQ):
            wo3[h] = stage[h * DH:(h + 1) * DH, :].astype(bf16)

        jcol = lax.broadcasted_iota(jnp.int32, (SQ, D), 1)
        pos = lax.broadcasted_iota(jnp.int32, (SQ, D), 0).astype(jnp.float32)
        d_in_head = jcol % DH
        expnt = (d_in_head - (d_in_head % 2)).astype(jnp.float32) * (1.0 / DH)
        ang = pos * jnp.exp(-math.log(10000.0) * expnt)
        cosb[...] = jnp.cos(ang).astype(bf16)
        sinb[...] = jnp.sin(ang).astype(bf16)
        even = (jcol % 2) == 0
        ones_col = jnp.ones((D, 1), bf16)

        acc = stage
        psend = xrecv

        def rope(t):
            t_r = jnp.where(even,
                            -jnp.roll(t, -1, axis=1),
                            jnp.roll(t, 1, axis=1))
            return t * cosb[...] + t_r * sinb[...]

        def compute_partial(xb):
            qf = rope(jnp.dot(xb, wqb[...],
                              preferred_element_type=jnp.float32).astype(bf16))
            kf = rope(jnp.dot(xb, wkb[...],
                              preferred_element_type=jnp.float32).astype(bf16))
            vf = jnp.dot(xb, wvb[...],
                         preferred_element_type=jnp.float32).astype(bf16)
            for h in range(HQ):
                q3[h] = qf[:, h * DH:(h + 1) * DH]
                k3[h] = kf[:, h * DH:(h + 1) * DH]
                v3[h] = vf[:, h * DH:(h + 1) * DH]

            acc[...] = jnp.zeros((SQ, D), jnp.float32)

            def head(h, carry):
                s = lax.dot_general(q3[h], k3[h], (((1,), (1,)), ((), ())),
                                    preferred_element_type=jnp.float32)
                w = jnp.exp(s.astype(bf16))
                denom = jnp.dot(w, ones_col,
                                preferred_element_type=jnp.float32)
                ctx = jnp.dot(w, v3[h],
                              preferred_element_type=jnp.float32) / denom
                acc[...] += jnp.dot(ctx.astype(bf16), wo3[h],
                                    preferred_element_type=jnp.float32)
                return carry

            lax.fori_loop(0, HQ, head, 0)

        p_rdmas = []
        for s in (1, 3, 2):
            k = s - 1
            pltpu.make_async_remote_copy(
                src_ref=xbf, dst_ref=xrecv.at[k],
                send_sem=xsend_sems.at[k], recv_sem=xrecv_sems.at[k],
                device_id=(i, ), device_id_type=pl.DeviceIdType.MESH,
            ).wait_recv()
            compute_partial(xrecv[k])
            psend[k] = acc[...].astype(bf16)
            rdma = pltpu.make_async_remote_copy(
                src_ref=psend.at[k],
                dst_ref=precv.at[3 - s],
                send_sem=psend_sems.at[k],
                recv_sem=precv_sems.at[3 - s],
                device_id=((i + s) % N_DEV,),
                device_id_type=pl.DeviceIdType.MESH,
            )
            rdma.start()
            p_rdmas.append(rdma)

        compute_partial(xbf[...])

        for k in range(3):
            pltpu.make_async_remote_copy(
                src_ref=psend.at[k], dst_ref=precv.at[k],
                send_sem=psend_sems.at[k], recv_sem=precv_sems.at[k],
                device_id=(i,), device_id_type=pl.DeviceIdType.MESH,
            ).wait_recv()
        acc[...] = (acc[...]
                    + precv[0].astype(jnp.float32)
                    + precv[1].astype(jnp.float32)
                    + precv[2].astype(jnp.float32))

        cp_out = pltpu.make_async_copy(acc, out_ref.at[0], stage_sem)
        cp_out.start()
        cp_out.wait()

        for rdma in x_rdmas + p_rdmas:
            rdma.wait_send()

    return pl.pallas_call(
        body,
        out_shape=jax.ShapeDtypeStruct((1, SQ, D), jnp.float32),
        in_specs=[pl.BlockSpec(memory_space=pl.ANY)] * 5,
        out_specs=pl.BlockSpec(memory_space=pltpu.MemorySpace.HBM),
        scratch_shapes=[
            pltpu.VMEM((SQ, D), jnp.float32),
            pltpu.VMEM((SQ, D), jnp.bfloat16),
            pltpu.VMEM((SQ, D), jnp.bfloat16),
            pltpu.VMEM((SQ, D), jnp.bfloat16),
            pltpu.VMEM((SQ, D), jnp.bfloat16),
            pltpu.VMEM((HQ, DH, D), jnp.bfloat16),
            pltpu.VMEM((SQ, D), jnp.bfloat16),
            pltpu.VMEM((SQ, D), jnp.bfloat16),
            pltpu.VMEM((HQ, SQ, DH), jnp.bfloat16),
            pltpu.VMEM((HQ, SQ, DH), jnp.bfloat16),
            pltpu.VMEM((HQ, SQ, DH), jnp.bfloat16),
            pltpu.VMEM((3, SQ, D), jnp.bfloat16),
            pltpu.VMEM((3, SQ, D), jnp.bfloat16),
            pltpu.SemaphoreType.DMA,
            pltpu.SemaphoreType.DMA((3,)),
            pltpu.SemaphoreType.DMA((3,)),
            pltpu.SemaphoreType.DMA((3,)),
            pltpu.SemaphoreType.DMA((3,)),
        ],
        compiler_params=pltpu.CompilerParams(
            collective_id=0,
            vmem_limit_bytes=100 * 1024 * 1024,
        ),
    )(x, Wq, Wk, Wv, Wo)


# device time: 160311 ns/iter; 1.3771x vs baseline; 1.0398x over previous
import math

import jax
import jax.numpy as jnp
from jax import lax
from jax.experimental import pallas as pl
from jax.experimental.pallas import tpu as pltpu

N_DEV = 4
SQ = 1024
D = 1024
HQ = 8
DH = 128
SCALE = 0.08838834764831843


def kernel(x, Wq, Wk, Wv, Wo):
    def body(x_hbm, wq_hbm, wk_hbm, wv_hbm, wo_hbm, out_ref,
             stage, xbf, wqb, wkb, wvb, wob, cosb, sinb,
             q3, k3, v3, c3, xrecv, precv,
             stage_sem, xsend_sems, xrecv_sems, psend_sems, precv_sems):
        i = lax.axis_index("i")
        bf16 = jnp.bfloat16

        cp_x = pltpu.make_async_copy(x_hbm.at[0], stage, stage_sem)
        cp_x.start()

        barrier = pltpu.get_barrier_semaphore()
        for s in (1, 2, 3):
            pl.semaphore_signal(
                barrier, inc=1,
                device_id=((i + s) % N_DEV,),
                device_id_type=pl.DeviceIdType.MESH,
            )
        pl.semaphore_wait(barrier, 3)

        cp_x.wait()
        xbf[...] = stage[...].astype(bf16)

        x_rdmas = []
        for s in (2, 1, 3):
            rdma = pltpu.make_async_remote_copy(
                src_ref=xbf,
                dst_ref=xrecv.at[3 - s],
                send_sem=xsend_sems.at[s - 1],
                recv_sem=xrecv_sems.at[3 - s],
                device_id=((i + s) % N_DEV,),
                device_id_type=pl.DeviceIdType.MESH,
            )
            rdma.start()
            x_rdmas.append(rdma)

        for w_hbm, wb, scl in ((wq_hbm, wqb, SCALE), (wk_hbm, wkb, 1.0),
                               (wv_hbm, wvb, 1.0)):
            cp_w = pltpu.make_async_copy(w_hbm, stage, stage_sem)
            cp_w.start()
            cp_w.wait()
            wb[...] = (stage[...] * scl).astype(bf16)
        cp_w = pltpu.make_async_copy(wo_hbm, stage, stage_sem)
        cp_w.start()
        cp_w.wait()
        wob[...] = stage[...].astype(bf16)

        jcol = lax.broadcasted_iota(jnp.int32, (SQ, D), 1)
        pos = lax.broadcasted_iota(jnp.int32, (SQ, D), 0).astype(jnp.float32)
        d_in_head = jcol % DH
        expnt = (d_in_head - (d_in_head % 2)).astype(jnp.float32) * (1.0 / DH)
        ang = pos * jnp.exp(-math.log(10000.0) * expnt)
        cosb[...] = jnp.cos(ang).astype(bf16)
        sinb[...] = jnp.sin(ang).astype(bf16)
        even = (jcol % 2) == 0
        ones_col = jnp.ones((D, 1), bf16)

        acc = stage
        psend = xrecv

        def rope(t):
            t_r = jnp.where(even,
                            -jnp.roll(t, -1, axis=1),
                            jnp.roll(t, 1, axis=1))
            return t * cosb[...] + t_r * sinb[...]

        def compute_partial(xb):
            qf = rope(jnp.dot(xb, wqb[...],
                              preferred_element_type=jnp.float32).astype(bf16))
            kf = rope(jnp.dot(xb, wkb[...],
                              preferred_element_type=jnp.float32).astype(bf16))
            vf = jnp.dot(xb, wvb[...],
                         preferred_element_type=jnp.float32).astype(bf16)
            for h in range(HQ):
                sl = slice(h * DH, (h + 1) * DH)
                q3[h] = qf[:, sl]
                k3[h] = kf[:, sl]
                v3[h] = vf[:, sl]

            def head(h, carry):
                s = lax.dot_general(q3[h], k3[h], (((1,), (1,)), ((), ())),
                                    preferred_element_type=jnp.float32)
                w = jnp.exp(s.astype(bf16))
                denom = jnp.dot(w, ones_col,
                                preferred_element_type=jnp.float32)
                ctx = jnp.dot(w, v3[h],
                              preferred_element_type=jnp.float32) / denom
                c3[h] = ctx.astype(bf16)
                return carry

            lax.fori_loop(0, HQ, head, 0)
            cfull = jnp.concatenate([c3[h] for h in range(HQ)], axis=1)
            acc[...] = jnp.dot(cfull, wob[...],
                               preferred_element_type=jnp.float32)

        p_rdmas = []
        for s in (1, 3, 2):
            k = s - 1
            pltpu.make_async_remote_copy(
                src_ref=xbf, dst_ref=xrecv.at[k],
                send_sem=xsend_sems.at[k], recv_sem=xrecv_sems.at[k],
                device_id=(i, ), device_id_type=pl.DeviceIdType.MESH,
            ).wait_recv()
            compute_partial(xrecv[k])
            psend[k] = acc[...].astype(bf16)
            rdma = pltpu.make_async_remote_copy(
                src_ref=psend.at[k],
                dst_ref=precv.at[3 - s],
                send_sem=psend_sems.at[k],
                recv_sem=precv_sems.at[3 - s],
                device_id=((i + s) % N_DEV,),
                device_id_type=pl.DeviceIdType.MESH,
            )
            rdma.start()
            p_rdmas.append(rdma)

        compute_partial(xbf[...])

        for k in range(3):
            pltpu.make_async_remote_copy(
                src_ref=psend.at[k], dst_ref=precv.at[k],
                send_sem=psend_sems.at[k], recv_sem=precv_sems.at[k],
                device_id=(i,), device_id_type=pl.DeviceIdType.MESH,
            ).wait_recv()
        acc[...] = (acc[...]
                    + precv[0].astype(jnp.float32)
                    + precv[1].astype(jnp.float32)
                    + precv[2].astype(jnp.float32))

        cp_out = pltpu.make_async_copy(acc, out_ref.at[0], stage_sem)
        cp_out.start()
        cp_out.wait()

        for rdma in x_rdmas + p_rdmas:
            rdma.wait_send()

    return pl.pallas_call(
        body,
        out_shape=jax.ShapeDtypeStruct((1, SQ, D), jnp.float32),
        in_specs=[pl.BlockSpec(memory_space=pl.ANY)] * 5,
        out_specs=pl.BlockSpec(memory_space=pltpu.MemorySpace.HBM),
        scratch_shapes=[
            pltpu.VMEM((SQ, D), jnp.float32),
            pltpu.VMEM((SQ, D), jnp.bfloat16),
            pltpu.VMEM((SQ, D), jnp.bfloat16),
            pltpu.VMEM((SQ, D), jnp.bfloat16),
            pltpu.VMEM((SQ, D), jnp.bfloat16),
            pltpu.VMEM((SQ, D), jnp.bfloat16),
            pltpu.VMEM((SQ, D), jnp.bfloat16),
            pltpu.VMEM((SQ, D), jnp.bfloat16),
            pltpu.VMEM((HQ, SQ, DH), jnp.bfloat16),
            pltpu.VMEM((HQ, SQ, DH), jnp.bfloat16),
            pltpu.VMEM((HQ, SQ, DH), jnp.bfloat16),
            pltpu.VMEM((HQ, SQ, DH), jnp.bfloat16),
            pltpu.VMEM((3, SQ, D), jnp.bfloat16),
            pltpu.VMEM((3, SQ, D), jnp.bfloat16),
            pltpu.SemaphoreType.DMA,
            pltpu.SemaphoreType.DMA((3,)),
            pltpu.SemaphoreType.DMA((3,)),
            pltpu.SemaphoreType.DMA((3,)),
            pltpu.SemaphoreType.DMA((3,)),
        ],
        compiler_params=pltpu.CompilerParams(
            collective_id=0,
            vmem_limit_bytes=100 * 1024 * 1024,
        ),
    )(x, Wq, Wk, Wv, Wo)
